# baseline (speedup 1.0000x reference)
import jax
import jax.numpy as jnp
from jax.experimental import pallas as pl


def _cumsum_lanes(x):
    n = x.shape[-1]
    shift = 1
    while shift < n:
        shifted = jnp.concatenate(
            [jnp.zeros(x.shape[:-1] + (shift,), x.dtype), x[..., :-shift]], axis=-1)
        x = x + shifted
        shift *= 2
    return x


_K = 10
_ROWS = 128
_H = 16
_C = 128


def _tc_kernel(embt_ref, emb_rows_ref, alpha_ref, phys_ref, out_ref):
    embt = embt_ref[0]
    emb_rows = emb_rows_ref[0]
    c = jax.nn.sigmoid(alpha_ref[0, 0])

    a = jax.lax.dot_general(emb_rows, embt, (((1,), (0,)), ((), ())),
                            preferred_element_type=jnp.float32)
    a = jnp.maximum(a, 0.0)
    r, n = a.shape
    ones = jnp.ones((n, 1), dtype=jnp.float32)

    def rowcount(mask):
        return jax.lax.dot_general(jnp.where(mask, 1.0, 0.0), ones,
                                   (((1,), (0,)), ((), ())),
                                   preferred_element_type=jnp.float32)

    cur = jnp.full((r, 1), jnp.inf, dtype=jnp.float32)
    thr = jnp.zeros((r, 1), dtype=jnp.float32)
    row_max = jnp.zeros((r, 1), dtype=jnp.float32)
    for k in range(_K):
        lt = a < cur
        n_ge = float(n) - rowcount(lt)
        d = jnp.max(jnp.where(lt, a, -1.0), axis=1, keepdims=True)
        take = n_ge < float(_K)
        thr = jnp.where(take, d, thr)
        if k == 0:
            row_max = d
        cur = d

    extra = float(_K) - rowcount(a > thr)

    eqf = jnp.where(a == thr, 1.0, 0.0)
    li = jax.lax.broadcasted_iota(jnp.int32, (_C, _C), 0)
    lj = jax.lax.broadcasted_iota(jnp.int32, (_C, _C), 1)
    tri = jnp.where(li <= lj, 1.0, 0.0).astype(jnp.float32)
    nc = n // _C
    parts = [jax.lax.dot_general(eqf[:, j * _C:(j + 1) * _C], tri,
                                 (((1,), (0,)), ((), ())),
                                 preferred_element_type=jnp.float32)
             for j in range(nc)]
    tot = jnp.concatenate([p[:, _C - 1:_C] for p in parts], axis=1)
    offs = _cumsum_lanes(tot) - tot
    rank = jnp.concatenate(
        [parts[j] + offs[:, j:j + 1] for j in range(nc)], axis=1)

    sel = jnp.where(
        jnp.logical_or(a > thr,
                       jnp.logical_and(a == thr, rank <= extra)), 1.0, 0.0)

    e = sel * jnp.exp(a - row_max)
    z = jnp.sum(e, axis=1, keepdims=True)
    phys = phys_ref[...]
    psum = jnp.sum(phys, axis=1, keepdims=True) + 1e-8
    out_ref[0, :, :] = (c / psum) * phys + ((1.0 - c) / z) * e


def kernel(x, A_physical, W, b, alpha):
    bsz, _, n, _ = x.shape
    state = x[:, -1, :, :]
    emb = jnp.tanh(state @ W + b)
    embt = jnp.swapaxes(emb, 1, 2)
    alpha2 = jnp.asarray(alpha, jnp.float32).reshape(1, 1)
    grid = (n // _ROWS, bsz)
    return pl.pallas_call(
        _tc_kernel,
        grid=grid,
        in_specs=[
            pl.BlockSpec((1, _H, n), lambda i, bb: (bb, 0, 0)),
            pl.BlockSpec((1, _ROWS, _H), lambda i, bb: (bb, i, 0)),
            pl.BlockSpec((1, 1), lambda i, bb: (0, 0)),
            pl.BlockSpec((_ROWS, n), lambda i, bb: (i, 0)),
        ],
        out_specs=pl.BlockSpec((1, _ROWS, n), lambda i, bb: (bb, i, 0)),
        out_shape=jax.ShapeDtypeStruct((bsz, n, n), jnp.float32),
    )(embt, emb, alpha2, A_physical)


# speedup vs baseline: 1.2056x; 1.2056x over previous
import jax
import jax.numpy as jnp
from jax.experimental import pallas as pl


def _cumsum_lanes(x):
    n = x.shape[-1]
    shift = 1
    while shift < n:
        shifted = jnp.concatenate(
            [jnp.zeros(x.shape[:-1] + (shift,), x.dtype), x[..., :-shift]], axis=-1)
        x = x + shifted
        shift *= 2
    return x


_K = 10
_ROWS = 512
_H = 16
_C = 128


def _tc_kernel(embt_ref, emb_rows_ref, alpha_ref, phys_ref, out_ref):
    embt = embt_ref[0]
    emb_rows = emb_rows_ref[0]
    c = jax.nn.sigmoid(alpha_ref[0, 0])

    a = jax.lax.dot_general(emb_rows, embt, (((1,), (0,)), ((), ())),
                            preferred_element_type=jnp.float32)
    a = jnp.maximum(a, 0.0)
    r, n = a.shape
    ones = jnp.ones((n, 1), dtype=jnp.float32)

    def rowcount(mask):
        return jax.lax.dot_general(jnp.where(mask, 1.0, 0.0), ones,
                                   (((1,), (0,)), ((), ())),
                                   preferred_element_type=jnp.float32)

    cur = jnp.full((r, 1), jnp.inf, dtype=jnp.float32)
    thr = jnp.zeros((r, 1), dtype=jnp.float32)
    row_max = jnp.zeros((r, 1), dtype=jnp.float32)
    for k in range(_K):
        lt = a < cur
        n_ge = float(n) - rowcount(lt)
        d = jnp.max(jnp.where(lt, a, -1.0), axis=1, keepdims=True)
        take = n_ge < float(_K)
        thr = jnp.where(take, d, thr)
        if k == 0:
            row_max = d
        cur = d

    extra = float(_K) - rowcount(a > thr)

    eqf = jnp.where(a == thr, 1.0, 0.0)
    li = jax.lax.broadcasted_iota(jnp.int32, (_C, _C), 0)
    lj = jax.lax.broadcasted_iota(jnp.int32, (_C, _C), 1)
    tri = jnp.where(li <= lj, 1.0, 0.0).astype(jnp.float32)
    nc = n // _C
    parts = [jax.lax.dot_general(eqf[:, j * _C:(j + 1) * _C], tri,
                                 (((1,), (0,)), ((), ())),
                                 preferred_element_type=jnp.float32)
             for j in range(nc)]
    tot = jnp.concatenate([p[:, _C - 1:_C] for p in parts], axis=1)
    offs = _cumsum_lanes(tot) - tot
    rank = jnp.concatenate(
        [parts[j] + offs[:, j:j + 1] for j in range(nc)], axis=1)

    e = jnp.where(
        jnp.logical_or(a > thr,
                       jnp.logical_and(a == thr, rank <= extra)),
        jnp.exp(a - row_max), 0.0)
    z = jnp.sum(e, axis=1, keepdims=True)
    phys = phys_ref[...]
    psum = jnp.sum(phys, axis=1, keepdims=True) + 1e-8
    out_ref[0, :, :] = (c / psum) * phys + ((1.0 - c) / z) * e


def kernel(x, A_physical, W, b, alpha):
    bsz, _, n, _ = x.shape
    state = x[:, -1, :, :]
    emb = jnp.tanh(state @ W + b)
    embt = jnp.swapaxes(emb, 1, 2)
    alpha2 = jnp.asarray(alpha, jnp.float32).reshape(1, 1)
    grid = (n // _ROWS, bsz)
    return pl.pallas_call(
        _tc_kernel,
        grid=grid,
        in_specs=[
            pl.BlockSpec((1, _H, n), lambda i, bb: (bb, 0, 0)),
            pl.BlockSpec((1, _ROWS, _H), lambda i, bb: (bb, i, 0)),
            pl.BlockSpec((1, 1), lambda i, bb: (0, 0)),
            pl.BlockSpec((_ROWS, n), lambda i, bb: (i, 0)),
        ],
        out_specs=pl.BlockSpec((1, _ROWS, n), lambda i, bb: (bb, i, 0)),
        out_shape=jax.ShapeDtypeStruct((bsz, n, n), jnp.float32),
    )(embt, emb, alpha2, A_physical)


# speedup vs baseline: 1.2204x; 1.0123x over previous
import jax
import jax.numpy as jnp
from jax.experimental import pallas as pl


def _cumsum_lanes(x):
    n = x.shape[-1]
    shift = 1
    while shift < n:
        shifted = jnp.concatenate(
            [jnp.zeros(x.shape[:-1] + (shift,), x.dtype), x[..., :-shift]], axis=-1)
        x = x + shifted
        shift *= 2
    return x


_K = 10
_ROWS = 512
_H = 16
_C = 128


def _tc_kernel(embt_ref, emb_rows_ref, alpha_ref, phys_ref, out_ref):
    embt = embt_ref[0]
    emb_rows = emb_rows_ref[0]
    c = jax.nn.sigmoid(alpha_ref[0, 0])

    a = jax.lax.dot_general(emb_rows, embt, (((1,), (0,)), ((), ())),
                            preferred_element_type=jnp.float32)
    a = jnp.maximum(a, 0.0)
    r, n = a.shape
    ones = jnp.ones((n, 1), dtype=jnp.float32)

    def rowcount(mask):
        return jax.lax.dot_general(jnp.where(mask, 1.0, 0.0), ones,
                                   (((1,), (0,)), ((), ())),
                                   preferred_element_type=jnp.float32)

    cur = jnp.full((r, 1), jnp.inf, dtype=jnp.float32)
    thr = jnp.zeros((r, 1), dtype=jnp.float32)
    row_max = jnp.zeros((r, 1), dtype=jnp.float32)
    for k in range(_K):
        lt = a < cur
        n_ge = float(n) - rowcount(lt)
        d = jnp.max(jnp.where(lt, a, -1.0), axis=1, keepdims=True)
        take = n_ge < float(_K)
        thr = jnp.where(take, d, thr)
        if k == 0:
            row_max = d
        cur = d

    extra = float(_K) - rowcount(a > thr)

    eqf = jnp.where(a == thr, 1.0, 0.0)
    li = jax.lax.broadcasted_iota(jnp.int32, (_C, _C), 0)
    lj = jax.lax.broadcasted_iota(jnp.int32, (_C, _C), 1)
    tri = jnp.where(li <= lj, 1.0, 0.0).astype(jnp.float32)
    nc = n // _C
    parts = [jax.lax.dot_general(eqf[:, j * _C:(j + 1) * _C], tri,
                                 (((1,), (0,)), ((), ())),
                                 preferred_element_type=jnp.float32)
             for j in range(nc)]
    tot = jnp.concatenate([p[:, _C - 1:_C] for p in parts], axis=1)
    offs = _cumsum_lanes(tot) - tot
    rank = jnp.concatenate(
        [parts[j] + offs[:, j:j + 1] for j in range(nc)], axis=1)

    sel = jnp.where(
        jnp.logical_or(a > thr,
                       jnp.logical_and(a == thr, rank <= extra)), 1.0, 0.0)

    e = sel * jnp.exp(a - row_max)
    z = jnp.sum(e, axis=1, keepdims=True)
    phys = phys_ref[...]
    psum = jnp.sum(phys, axis=1, keepdims=True) + 1e-8
    out_ref[0, :, :] = (c / psum) * phys + ((1.0 - c) / z) * e


def kernel(x, A_physical, W, b, alpha):
    bsz, _, n, _ = x.shape
    state = x[:, -1, :, :]
    emb = jnp.tanh(state @ W + b)
    embt = jnp.swapaxes(emb, 1, 2)
    alpha2 = jnp.asarray(alpha, jnp.float32).reshape(1, 1)
    grid = (n // _ROWS, bsz)
    return pl.pallas_call(
        _tc_kernel,
        grid=grid,
        in_specs=[
            pl.BlockSpec((1, _H, n), lambda i, bb: (bb, 0, 0)),
            pl.BlockSpec((1, _ROWS, _H), lambda i, bb: (bb, i, 0)),
            pl.BlockSpec((1, 1), lambda i, bb: (0, 0)),
            pl.BlockSpec((_ROWS, n), lambda i, bb: (i, 0)),
        ],
        out_specs=pl.BlockSpec((1, _ROWS, n), lambda i, bb: (bb, i, 0)),
        out_shape=jax.ShapeDtypeStruct((bsz, n, n), jnp.float32),
    )(embt, emb, alpha2, A_physical)


# speedup vs baseline: 1.2221x; 1.0014x over previous
import jax
import jax.numpy as jnp
from jax.experimental import pallas as pl


def _cumsum_lanes(x):
    n = x.shape[-1]
    shift = 1
    while shift < n:
        shifted = jnp.concatenate(
            [jnp.zeros(x.shape[:-1] + (shift,), x.dtype), x[..., :-shift]], axis=-1)
        x = x + shifted
        shift *= 2
    return x


_K = 10
_ROWS = 512
_H = 16
_C = 128


def _tc_kernel(embt_ref, emb_rows_ref, alpha_ref, phys_ref, out_ref):
    embt = embt_ref[0]
    emb_rows = emb_rows_ref[0]
    c = jax.nn.sigmoid(alpha_ref[0, 0])

    a = jax.lax.dot_general(emb_rows, embt, (((1,), (0,)), ((), ())),
                            preferred_element_type=jnp.float32)
    a = jnp.maximum(a, 0.0)
    r, n = a.shape
    ones = jnp.ones((n, 1), dtype=jnp.float32)

    def rowcount(mask):
        return jax.lax.dot_general(jnp.where(mask, 1.0, 0.0), ones,
                                   (((1,), (0,)), ((), ())),
                                   preferred_element_type=jnp.float32)

    cur = jnp.full((r, 1), jnp.inf, dtype=jnp.float32)
    thr = jnp.zeros((r, 1), dtype=jnp.float32)
    row_max = jnp.zeros((r, 1), dtype=jnp.float32)
    for k in range(_K):
        lt = a < cur
        n_ge = float(n) - rowcount(lt)
        d = jnp.max(jnp.where(lt, a, -1.0), axis=1, keepdims=True)
        take = n_ge < float(_K)
        thr = jnp.where(take, d, thr)
        if k == 0:
            row_max = d
        cur = d

    extra = float(_K) - rowcount(a > thr)

    eqf = jnp.where(a == thr, 1.0, 0.0)
    li = jax.lax.broadcasted_iota(jnp.int32, (_C, _C), 0)
    lj = jax.lax.broadcasted_iota(jnp.int32, (_C, _C), 1)
    tri = jnp.where(li <= lj, 1.0, 0.0).astype(jnp.float32)
    nc = n // _C
    parts = [jax.lax.dot_general(eqf[:, j * _C:(j + 1) * _C], tri,
                                 (((1,), (0,)), ((), ())),
                                 preferred_element_type=jnp.float32)
             for j in range(nc)]
    tot = jnp.concatenate([p[:, _C - 1:_C] for p in parts], axis=1)
    offs = _cumsum_lanes(tot) - tot
    rank = jnp.concatenate(
        [parts[j] + offs[:, j:j + 1] for j in range(nc)], axis=1)

    sel = jnp.where(
        jnp.logical_or(a > thr,
                       jnp.logical_and(a == thr, rank <= extra)), 1.0, 0.0)

    e = sel * jnp.exp(a - row_max)
    z = jax.lax.dot_general(e, ones, (((1,), (0,)), ((), ())),
                            preferred_element_type=jnp.float32)
    phys = phys_ref[...]
    psum = jax.lax.dot_general(phys, ones, (((1,), (0,)), ((), ())),
                               preferred_element_type=jnp.float32) + 1e-8
    out_ref[0, :, :] = (c / psum) * phys + ((1.0 - c) / z) * e


def kernel(x, A_physical, W, b, alpha):
    bsz, _, n, _ = x.shape
    state = x[:, -1, :, :]
    emb = jnp.tanh(state @ W + b)
    embt = jnp.swapaxes(emb, 1, 2)
    alpha2 = jnp.asarray(alpha, jnp.float32).reshape(1, 1)
    grid = (n // _ROWS, bsz)
    return pl.pallas_call(
        _tc_kernel,
        grid=grid,
        in_specs=[
            pl.BlockSpec((1, _H, n), lambda i, bb: (bb, 0, 0)),
            pl.BlockSpec((1, _ROWS, _H), lambda i, bb: (bb, i, 0)),
            pl.BlockSpec((1, 1), lambda i, bb: (0, 0)),
            pl.BlockSpec((_ROWS, n), lambda i, bb: (i, 0)),
        ],
        out_specs=pl.BlockSpec((1, _ROWS, n), lambda i, bb: (bb, i, 0)),
        out_shape=jax.ShapeDtypeStruct((bsz, n, n), jnp.float32),
    )(embt, emb, alpha2, A_physical)
